# Initial kernel scaffold; baseline (speedup 1.0000x reference)
import jax
import jax.numpy as jnp
from jax.experimental import pallas as pl


def kernel(inputs, graph, W0_self, W0_neigh, b0, W1_self, W1_neigh, b1):
    raise NotImplementedError("write your pallas kernel here")


# speedup vs baseline: 6.9880x; 6.9880x over previous
import jax
import jax.numpy as jnp
from jax import lax
from jax.experimental import pallas as pl
from jax.experimental.pallas import tpu as pltpu
from jax.experimental.pallas import tpu_sc as plsc

BLK = 1000


def _sc_edge_scatter(g, src, dst):
    n, w = g.shape
    e = src.shape[0]
    mesh = plsc.VectorSubcoreMesh(core_axis_name="c", subcore_axis_name="s")
    nc, ns = mesh.num_cores, mesh.num_subcores
    nw = nc * ns
    epw = e // nw
    zr = k = 80
    nch = epw // k
    nzch = n // zr
    jmax = (nzch + ns - 1) // ns
    wl = w // 16
    assert e % nw == 0 and epw % k == 0 and n % zr == 0

    def body(g_hbm, src_hbm, dst_hbm, out_acc, acc_sh, sidx, didx, rows,
             semi, semg):
        cid = lax.axis_index("c")
        sid = lax.axis_index("s")
        wid = cid * ns + sid
        base = wid * epw
        zv = jnp.zeros((16,), jnp.float32)

        def zfill(i, _):
            rows[0, i // wl, pl.ds((i % wl) * 16, 16)] = zv
            return 0

        lax.fori_loop(0, zr * wl, zfill, 0)
        for j in range(jmax):
            c = sid + j * ns

            @pl.when(c < nzch)
            def _():
                pltpu.sync_copy(rows.at[0], acc_sh.at[pl.ds(c * zr, zr)])
        plsc.subcore_barrier()

        def stage(c, b):
            pltpu.async_copy(src_hbm.at[pl.ds(base + c * k, k)], sidx.at[b], semi)
            pltpu.async_copy(dst_hbm.at[pl.ds(base + c * k, k)], didx.at[b], semi)

        def stage_wait(c, b):
            pltpu.make_async_copy(
                src_hbm.at[pl.ds(base + c * k, k)], sidx.at[b], semi).wait()
            pltpu.make_async_copy(
                dst_hbm.at[pl.ds(base + c * k, k)], didx.at[b], semi).wait()

        def gath(c, b):
            pltpu.async_copy(g_hbm.at[sidx.at[b]], rows.at[b], semg)

        def gath_wait(c, b):
            pltpu.make_async_copy(g_hbm.at[sidx.at[b]], rows.at[b], semg).wait()

        stage(0, 0)

        @pl.when(1 < nch)
        def _():
            stage(1, 1)

        stage_wait(0, 0)
        gath(0, 0)

        def chunk(c, b):
            gath_wait(c, b)
            pltpu.sync_copy(rows.at[b], acc_sh.at[didx.at[b]], add=True)

            @pl.when(c + 2 < nch)
            def _():
                stage(c + 2, b)

            @pl.when(c + 1 < nch)
            def _():
                stage_wait(c + 1, 1 - b)
                gath(c + 1, 1 - b)

        def loop2(i, _):
            c0 = i * 2

            @pl.when(c0 < nch)
            def _():
                chunk(c0, 0)

            @pl.when(c0 + 1 < nch)
            def _():
                chunk(c0 + 1, 1)

            return 0

        lax.fori_loop(0, (nch + 1) // 2, loop2, 0)
        plsc.subcore_barrier()
        for j in range(jmax):
            c = sid + j * ns

            @pl.when(c < nzch)
            def _():
                pltpu.sync_copy(acc_sh.at[pl.ds(c * zr, zr)],
                                out_acc.at[cid, pl.ds(c * zr, zr)])

    f = pl.kernel(
        body,
        out_type=jax.ShapeDtypeStruct((nc, n, w), jnp.float32),
        mesh=mesh,
        scratch_types=[
            pltpu.VMEM_SHARED((n, w), jnp.float32),
            pltpu.VMEM((2, k), jnp.int32),
            pltpu.VMEM((2, k), jnp.int32),
            pltpu.VMEM((2, k, w), jnp.float32),
            pltpu.SemaphoreType.DMA,
            pltpu.SemaphoreType.DMA,
        ],
    )
    return f(g, src, dst)


def _sc_degree(dst, n):
    e = dst.shape[0]
    w = 128
    mesh = plsc.VectorSubcoreMesh(core_axis_name="c", subcore_axis_name="s")
    nc, ns = mesh.num_cores, mesh.num_subcores
    nw = nc * ns
    epw = e // nw
    zr = k = 80
    nch = epw // k
    nzch = n // zr
    jmax = (nzch + ns - 1) // ns
    wl = w // 16
    assert e % nw == 0 and epw % k == 0 and n % zr == 0

    def body(dst_hbm, out_deg, deg_sh, didx, onesb, semi):
        cid = lax.axis_index("c")
        sid = lax.axis_index("s")
        wid = cid * ns + sid
        base = wid * epw
        zv = jnp.zeros((16,), jnp.float32)
        ov = jnp.ones((16,), jnp.float32)

        def zfill(i, _):
            onesb[i // wl, pl.ds((i % wl) * 16, 16)] = zv
            return 0

        lax.fori_loop(0, k * wl, zfill, 0)
        for j in range(jmax):
            c = sid + j * ns

            @pl.when(c < nzch)
            def _():
                pltpu.sync_copy(onesb, deg_sh.at[pl.ds(c * zr, zr)])

        def ofill(i, _):
            onesb[i // wl, pl.ds((i % wl) * 16, 16)] = ov
            return 0

        lax.fori_loop(0, k * wl, ofill, 0)
        plsc.subcore_barrier()

        def stage(c, b):
            pltpu.async_copy(dst_hbm.at[pl.ds(base + c * k, k)], didx.at[b], semi)

        def stage_wait(c, b):
            pltpu.make_async_copy(
                dst_hbm.at[pl.ds(base + c * k, k)], didx.at[b], semi).wait()

        stage(0, 0)

        def chunk(c, b):
            stage_wait(c, b)

            @pl.when(c + 1 < nch)
            def _():
                stage(c + 1, 1 - b)

            pltpu.sync_copy(onesb, deg_sh.at[didx.at[b]], add=True)

        def loop2(i, _):
            c0 = i * 2

            @pl.when(c0 < nch)
            def _():
                chunk(c0, 0)

            @pl.when(c0 + 1 < nch)
            def _():
                chunk(c0 + 1, 1)

            return 0

        lax.fori_loop(0, (nch + 1) // 2, loop2, 0)
        plsc.subcore_barrier()
        for j in range(jmax):
            c = sid + j * ns

            @pl.when(c < nzch)
            def _():
                pltpu.sync_copy(deg_sh.at[pl.ds(c * zr, zr)],
                                out_deg.at[cid, pl.ds(c * zr, zr)])

    f = pl.kernel(
        body,
        out_type=jax.ShapeDtypeStruct((nc, n, w), jnp.float32),
        mesh=mesh,
        scratch_types=[
            pltpu.VMEM_SHARED((n, w), jnp.float32),
            pltpu.VMEM((2, k), jnp.int32),
            pltpu.VMEM((k, w), jnp.float32),
            pltpu.SemaphoreType.DMA,
        ],
    )
    return f(dst)


def _mm_dual_body(x_ref, wa_ref, wb_ref, a_ref, b_ref):
    x = x_ref[...]
    a_ref[...] = jnp.dot(x, wa_ref[...], preferred_element_type=jnp.float32)
    b_ref[...] = jnp.dot(x, wb_ref[...], preferred_element_type=jnp.float32)


def _mm_dual(x, wa, wb):
    n, d = x.shape
    ha, hb = wa.shape[1], wb.shape[1]
    return pl.pallas_call(
        _mm_dual_body,
        grid=(n // BLK,),
        in_specs=[
            pl.BlockSpec((BLK, d), lambda i: (i, 0)),
            pl.BlockSpec((d, ha), lambda i: (0, 0)),
            pl.BlockSpec((d, hb), lambda i: (0, 0)),
        ],
        out_specs=[
            pl.BlockSpec((BLK, ha), lambda i: (i, 0)),
            pl.BlockSpec((BLK, hb), lambda i: (i, 0)),
        ],
        out_shape=[
            jax.ShapeDtypeStruct((n, ha), jnp.float32),
            jax.ShapeDtypeStruct((n, hb), jnp.float32),
        ],
    )(x, wa, wb)


def _invdeg_body(degp_ref, out_ref):
    d = degp_ref[0] + degp_ref[1]
    e0 = (lax.broadcasted_iota(jnp.int32, (d.shape[1], 1), 0) == 0)
    deg = jnp.dot(d, e0.astype(jnp.float32),
                  preferred_element_type=jnp.float32)
    out_ref[...] = 1.0 / jnp.maximum(deg, 1.0)


def _invdeg(degp):
    nc, n, w = degp.shape
    return pl.pallas_call(
        _invdeg_body,
        grid=(n // BLK,),
        in_specs=[pl.BlockSpec((nc, BLK, w), lambda i: (0, i, 0))],
        out_specs=pl.BlockSpec((BLK, 1), lambda i: (i, 0)),
        out_shape=jax.ShapeDtypeStruct((n, 1), jnp.float32),
    )(degp)


def _mid_body(self0_ref, s0p_ref, invd_ref, b0_ref, h1_ref):
    s0 = s0p_ref[0] + s0p_ref[1]
    h1_ref[...] = jnp.maximum(
        self0_ref[...] + s0 * invd_ref[...] + b0_ref[...], 0.0)


def _mid(self0, s0p, invd, b0):
    n, h = self0.shape
    nc = s0p.shape[0]
    return pl.pallas_call(
        _mid_body,
        grid=(n // BLK,),
        in_specs=[
            pl.BlockSpec((BLK, h), lambda i: (i, 0)),
            pl.BlockSpec((nc, BLK, h), lambda i: (0, i, 0)),
            pl.BlockSpec((BLK, 1), lambda i: (i, 0)),
            pl.BlockSpec((1, h), lambda i: (0, 0)),
        ],
        out_specs=pl.BlockSpec((BLK, h), lambda i: (i, 0)),
        out_shape=jax.ShapeDtypeStruct((n, h), jnp.float32),
    )(self0, s0p, invd, b0)


def _final_body(h1_ref, s1p_ref, invd_ref, b1_ref, w1s_ref, w1n_ref, out_ref):
    hn = (s1p_ref[0] + s1p_ref[1]) * invd_ref[...]
    out_ref[...] = (
        jnp.dot(h1_ref[...], w1s_ref[...], preferred_element_type=jnp.float32)
        + jnp.dot(hn, w1n_ref[...], preferred_element_type=jnp.float32)
        + b1_ref[...])


def _final(h1, s1p, invd, b1, w1s, w1n):
    n, h = h1.shape
    nc = s1p.shape[0]
    c = w1s.shape[1]
    return pl.pallas_call(
        _final_body,
        grid=(n // BLK,),
        in_specs=[
            pl.BlockSpec((BLK, h), lambda i: (i, 0)),
            pl.BlockSpec((nc, BLK, h), lambda i: (0, i, 0)),
            pl.BlockSpec((BLK, 1), lambda i: (i, 0)),
            pl.BlockSpec((1, c), lambda i: (0, 0)),
            pl.BlockSpec((h, c), lambda i: (0, 0)),
            pl.BlockSpec((h, c), lambda i: (0, 0)),
        ],
        out_specs=pl.BlockSpec((BLK, c), lambda i: (i, 0)),
        out_shape=jax.ShapeDtypeStruct((n, c), jnp.float32),
    )(h1, s1p, invd, b1, w1s, w1n)


def kernel(inputs, graph, W0_self, W0_neigh, b0, W1_self, W1_neigh, b1):
    n = inputs.shape[0]
    gi = graph.astype(jnp.int32)
    src, dst = gi[0], gi[1]
    degp = _sc_degree(dst, n)
    invd = _invdeg(degp)
    self0, g0 = _mm_dual(inputs, W0_self, W0_neigh)
    s0p = _sc_edge_scatter(g0, src, dst)
    h1 = _mid(self0, s0p, invd, b0.reshape(1, -1))
    s1p = _sc_edge_scatter(h1, src, dst)
    return _final(h1, s1p, invd, b1.reshape(1, -1), W1_self, W1_neigh)


# speedup vs baseline: 8.5212x; 1.2194x over previous
import jax
import jax.numpy as jnp
from jax import lax
from jax.experimental import pallas as pl
from jax.experimental.pallas import tpu as pltpu
from jax.experimental.pallas import tpu_sc as plsc

BLK = 1000


def _sc_edge_scatter(g, src, dst):
    n, w = g.shape
    e = src.shape[0]
    mesh = plsc.VectorSubcoreMesh(core_axis_name="c", subcore_axis_name="s")
    nc, ns = mesh.num_cores, mesh.num_subcores
    nw = nc * ns
    epw = e // nw
    zr = k = 80
    nch = epw // k
    nzch = n // zr
    jmax = (nzch + ns - 1) // ns
    wl = w // 16
    assert e % nw == 0 and epw % k == 0 and n % zr == 0

    def body(g_hbm, src_hbm, dst_hbm, out_acc, acc_sh, sidx, didx, didx2,
             rows, semi, semg, sems):
        cid = lax.axis_index("c")
        sid = lax.axis_index("s")
        wid = cid * ns + sid
        base = wid * epw
        zv = jnp.zeros((16,), jnp.float32)

        def zfill(i, _):
            rows[0, i // wl, pl.ds((i % wl) * 16, 16)] = zv
            return 0

        lax.fori_loop(0, zr * wl, zfill, 0)
        for j in range(jmax):
            c = sid + j * ns

            @pl.when(c < nzch)
            def _():
                pltpu.sync_copy(rows.at[0], acc_sh.at[pl.ds(c * zr, zr)])
        plsc.subcore_barrier()

        def stage(c, b):
            pltpu.async_copy(src_hbm.at[pl.ds(base + c * k, k)], sidx.at[b], semi)
            pltpu.async_copy(dst_hbm.at[pl.ds(base + c * k, k)], didx.at[b], semi)

        def stage_wait(c, b):
            pltpu.make_async_copy(
                src_hbm.at[pl.ds(base + c * k, k)], sidx.at[b], semi).wait()
            pltpu.make_async_copy(
                dst_hbm.at[pl.ds(base + c * k, k)], didx.at[b], semi).wait()

        def gath(c, b):
            pltpu.async_copy(g_hbm.at[sidx.at[b]], rows.at[b], semg)

        def gath_wait(c, b):
            pltpu.make_async_copy(g_hbm.at[sidx.at[b]], rows.at[b], semg).wait()

        stage(0, 0)

        @pl.when(1 < nch)
        def _():
            stage(1, 1)

        stage_wait(0, 0)
        gath(0, 0)

        def scat_wait(b):
            pltpu.make_async_copy(
                rows.at[b], acc_sh.at[didx2.at[b]], sems).wait()

        def chunk(c, b):
            gath_wait(c, b)
            for j in range(k // 16):
                didx2[b, pl.ds(j * 16, 16)] = didx[b, pl.ds(j * 16, 16)]
            pltpu.async_copy(rows.at[b], acc_sh.at[didx2.at[b]], sems,
                             add=True)

            @pl.when(c + 2 < nch)
            def _():
                stage(c + 2, b)

            @pl.when(c >= 1)
            def _():
                scat_wait(1 - b)

            @pl.when(c + 1 < nch)
            def _():
                stage_wait(c + 1, 1 - b)
                gath(c + 1, 1 - b)

        def loop2(i, _):
            c0 = i * 2

            @pl.when(c0 < nch)
            def _():
                chunk(c0, 0)

            @pl.when(c0 + 1 < nch)
            def _():
                chunk(c0 + 1, 1)

            return 0

        lax.fori_loop(0, (nch + 1) // 2, loop2, 0)
        scat_wait((nch - 1) % 2)
        plsc.subcore_barrier()
        for j in range(jmax):
            c = sid + j * ns

            @pl.when(c < nzch)
            def _():
                pltpu.sync_copy(acc_sh.at[pl.ds(c * zr, zr)],
                                out_acc.at[cid, pl.ds(c * zr, zr)])

    f = pl.kernel(
        body,
        out_type=jax.ShapeDtypeStruct((nc, n, w), jnp.float32),
        mesh=mesh,
        scratch_types=[
            pltpu.VMEM_SHARED((n, w), jnp.float32),
            pltpu.VMEM((2, k), jnp.int32),
            pltpu.VMEM((2, k), jnp.int32),
            pltpu.VMEM((2, k), jnp.int32),
            pltpu.VMEM((2, k, w), jnp.float32),
            pltpu.SemaphoreType.DMA,
            pltpu.SemaphoreType.DMA,
            pltpu.SemaphoreType.DMA,
        ],
    )
    return f(g, src, dst)


def _sc_degree(dst, n):
    e = dst.shape[0]
    w = 128
    mesh = plsc.VectorSubcoreMesh(core_axis_name="c", subcore_axis_name="s")
    nc, ns = mesh.num_cores, mesh.num_subcores
    nw = nc * ns
    epw = e // nw
    zr = k = 80
    nch = epw // k
    nzch = n // zr
    jmax = (nzch + ns - 1) // ns
    wl = w // 16
    assert e % nw == 0 and epw % k == 0 and n % zr == 0

    def body(dst_hbm, out_deg, deg_sh, didx, onesb, semi):
        cid = lax.axis_index("c")
        sid = lax.axis_index("s")
        wid = cid * ns + sid
        base = wid * epw
        zv = jnp.zeros((16,), jnp.float32)
        ov = jnp.ones((16,), jnp.float32)

        def zfill(i, _):
            onesb[i // wl, pl.ds((i % wl) * 16, 16)] = zv
            return 0

        lax.fori_loop(0, k * wl, zfill, 0)
        for j in range(jmax):
            c = sid + j * ns

            @pl.when(c < nzch)
            def _():
                pltpu.sync_copy(onesb, deg_sh.at[pl.ds(c * zr, zr)])

        def ofill(i, _):
            onesb[i // wl, pl.ds((i % wl) * 16, 16)] = ov
            return 0

        lax.fori_loop(0, k * wl, ofill, 0)
        plsc.subcore_barrier()

        def stage(c, b):
            pltpu.async_copy(dst_hbm.at[pl.ds(base + c * k, k)], didx.at[b], semi)

        def stage_wait(c, b):
            pltpu.make_async_copy(
                dst_hbm.at[pl.ds(base + c * k, k)], didx.at[b], semi).wait()

        stage(0, 0)

        def chunk(c, b):
            stage_wait(c, b)

            @pl.when(c + 1 < nch)
            def _():
                stage(c + 1, 1 - b)

            pltpu.sync_copy(onesb, deg_sh.at[didx.at[b]], add=True)

        def loop2(i, _):
            c0 = i * 2

            @pl.when(c0 < nch)
            def _():
                chunk(c0, 0)

            @pl.when(c0 + 1 < nch)
            def _():
                chunk(c0 + 1, 1)

            return 0

        lax.fori_loop(0, (nch + 1) // 2, loop2, 0)
        plsc.subcore_barrier()
        for j in range(jmax):
            c = sid + j * ns

            @pl.when(c < nzch)
            def _():
                pltpu.sync_copy(deg_sh.at[pl.ds(c * zr, zr)],
                                out_deg.at[cid, pl.ds(c * zr, zr)])

    f = pl.kernel(
        body,
        out_type=jax.ShapeDtypeStruct((nc, n, w), jnp.float32),
        mesh=mesh,
        scratch_types=[
            pltpu.VMEM_SHARED((n, w), jnp.float32),
            pltpu.VMEM((2, k), jnp.int32),
            pltpu.VMEM((k, w), jnp.float32),
            pltpu.SemaphoreType.DMA,
        ],
    )
    return f(dst)


def _mm_dual_body(x_ref, wa_ref, wb_ref, a_ref, b_ref):
    x = x_ref[...]
    a_ref[...] = jnp.dot(x, wa_ref[...], preferred_element_type=jnp.float32)
    b_ref[...] = jnp.dot(x, wb_ref[...], preferred_element_type=jnp.float32)


def _mm_dual(x, wa, wb):
    n, d = x.shape
    ha, hb = wa.shape[1], wb.shape[1]
    return pl.pallas_call(
        _mm_dual_body,
        grid=(n // BLK,),
        in_specs=[
            pl.BlockSpec((BLK, d), lambda i: (i, 0)),
            pl.BlockSpec((d, ha), lambda i: (0, 0)),
            pl.BlockSpec((d, hb), lambda i: (0, 0)),
        ],
        out_specs=[
            pl.BlockSpec((BLK, ha), lambda i: (i, 0)),
            pl.BlockSpec((BLK, hb), lambda i: (i, 0)),
        ],
        out_shape=[
            jax.ShapeDtypeStruct((n, ha), jnp.float32),
            jax.ShapeDtypeStruct((n, hb), jnp.float32),
        ],
    )(x, wa, wb)


def _invdeg_body(degp_ref, out_ref):
    d = degp_ref[0] + degp_ref[1]
    e0 = (lax.broadcasted_iota(jnp.int32, (d.shape[1], 1), 0) == 0)
    deg = jnp.dot(d, e0.astype(jnp.float32),
                  preferred_element_type=jnp.float32)
    out_ref[...] = 1.0 / jnp.maximum(deg, 1.0)


def _invdeg(degp):
    nc, n, w = degp.shape
    return pl.pallas_call(
        _invdeg_body,
        grid=(n // BLK,),
        in_specs=[pl.BlockSpec((nc, BLK, w), lambda i: (0, i, 0))],
        out_specs=pl.BlockSpec((BLK, 1), lambda i: (i, 0)),
        out_shape=jax.ShapeDtypeStruct((n, 1), jnp.float32),
    )(degp)


def _mid_body(self0_ref, s0p_ref, invd_ref, b0_ref, h1_ref):
    s0 = s0p_ref[0] + s0p_ref[1]
    h1_ref[...] = jnp.maximum(
        self0_ref[...] + s0 * invd_ref[...] + b0_ref[...], 0.0)


def _mid(self0, s0p, invd, b0):
    n, h = self0.shape
    nc = s0p.shape[0]
    return pl.pallas_call(
        _mid_body,
        grid=(n // BLK,),
        in_specs=[
            pl.BlockSpec((BLK, h), lambda i: (i, 0)),
            pl.BlockSpec((nc, BLK, h), lambda i: (0, i, 0)),
            pl.BlockSpec((BLK, 1), lambda i: (i, 0)),
            pl.BlockSpec((1, h), lambda i: (0, 0)),
        ],
        out_specs=pl.BlockSpec((BLK, h), lambda i: (i, 0)),
        out_shape=jax.ShapeDtypeStruct((n, h), jnp.float32),
    )(self0, s0p, invd, b0)


def _final_body(h1_ref, s1p_ref, invd_ref, b1_ref, w1s_ref, w1n_ref, out_ref):
    hn = (s1p_ref[0] + s1p_ref[1]) * invd_ref[...]
    out_ref[...] = (
        jnp.dot(h1_ref[...], w1s_ref[...], preferred_element_type=jnp.float32)
        + jnp.dot(hn, w1n_ref[...], preferred_element_type=jnp.float32)
        + b1_ref[...])


def _final(h1, s1p, invd, b1, w1s, w1n):
    n, h = h1.shape
    nc = s1p.shape[0]
    c = w1s.shape[1]
    return pl.pallas_call(
        _final_body,
        grid=(n // BLK,),
        in_specs=[
            pl.BlockSpec((BLK, h), lambda i: (i, 0)),
            pl.BlockSpec((nc, BLK, h), lambda i: (0, i, 0)),
            pl.BlockSpec((BLK, 1), lambda i: (i, 0)),
            pl.BlockSpec((1, c), lambda i: (0, 0)),
            pl.BlockSpec((h, c), lambda i: (0, 0)),
            pl.BlockSpec((h, c), lambda i: (0, 0)),
        ],
        out_specs=pl.BlockSpec((BLK, c), lambda i: (i, 0)),
        out_shape=jax.ShapeDtypeStruct((n, c), jnp.float32),
    )(h1, s1p, invd, b1, w1s, w1n)


def kernel(inputs, graph, W0_self, W0_neigh, b0, W1_self, W1_neigh, b1):
    n = inputs.shape[0]
    gi = graph.astype(jnp.int32)
    src, dst = gi[0], gi[1]
    degp = _sc_degree(dst, n)
    invd = _invdeg(degp)
    self0, g0 = _mm_dual(inputs, W0_self, W0_neigh)
    s0p = _sc_edge_scatter(g0, src, dst)
    h1 = _mid(self0, s0p, invd, b0.reshape(1, -1))
    s1p = _sc_edge_scatter(h1, src, dst)
    return _final(h1, s1p, invd, b1.reshape(1, -1), W1_self, W1_neigh)


# speedup vs baseline: 11.0759x; 1.2998x over previous
import jax
import jax.numpy as jnp
from jax import lax
from jax.experimental import pallas as pl
from jax.experimental.pallas import tpu as pltpu
from jax.experimental.pallas import tpu_sc as plsc

BLK = 1000


def _sc_edge_scatter(g, src, dst):
    n, w = g.shape
    e = src.shape[0]
    mesh = plsc.VectorSubcoreMesh(core_axis_name="c", subcore_axis_name="s")
    nc, ns = mesh.num_cores, mesh.num_subcores
    nw = nc * ns
    epw = e // nw
    zr = k = 80
    nch = epw // k
    nzch = n // zr
    jmax = (nzch + ns - 1) // ns
    wl = w // 16
    assert e % nw == 0 and epw % k == 0 and n % zr == 0

    def body(g_hbm, src_hbm, dst_hbm, out_acc, acc_sh, sidx, didx, didx2,
             rows, semi, semg, sems):
        cid = lax.axis_index("c")
        sid = lax.axis_index("s")
        wid = cid * ns + sid
        base = wid * epw
        zv = jnp.zeros((16,), jnp.float32)

        def zfill(i, _):
            rows[0, i // wl, pl.ds((i % wl) * 16, 16)] = zv
            return 0

        lax.fori_loop(0, zr * wl, zfill, 0)
        for j in range(jmax):
            c = sid + j * ns

            @pl.when(c < nzch)
            def _():
                pltpu.sync_copy(rows.at[0], acc_sh.at[pl.ds(c * zr, zr)])
        plsc.subcore_barrier()

        def stage(c, b):
            pltpu.async_copy(src_hbm.at[pl.ds(base + c * k, k)], sidx.at[b], semi)
            pltpu.async_copy(dst_hbm.at[pl.ds(base + c * k, k)], didx.at[b], semi)

        def stage_wait(c, b):
            pltpu.make_async_copy(
                src_hbm.at[pl.ds(base + c * k, k)], sidx.at[b], semi).wait()
            pltpu.make_async_copy(
                dst_hbm.at[pl.ds(base + c * k, k)], didx.at[b], semi).wait()

        def gath(c, b):
            pltpu.async_copy(g_hbm.at[sidx.at[b]], rows.at[b], semg)

        def gath_wait(c, b):
            pltpu.make_async_copy(g_hbm.at[sidx.at[b]], rows.at[b], semg).wait()

        nb = 3
        stage(0, 0)
        stage(1, 1)
        stage(2, 2)
        stage_wait(0, 0)
        gath(0, 0)
        stage_wait(1, 1)
        gath(1, 1)

        def scat_wait(b):
            pltpu.make_async_copy(
                rows.at[b], acc_sh.at[didx2.at[b]], sems).wait()

        def chunk(c, b):
            gath_wait(c, b)
            for j in range(k // 16):
                didx2[b, pl.ds(j * 16, 16)] = didx[b, pl.ds(j * 16, 16)]
            pltpu.async_copy(rows.at[b], acc_sh.at[didx2.at[b]], sems,
                             add=True)

            @pl.when(c + nb < nch)
            def _():
                stage(c + nb, b)

            @pl.when(c >= 1)
            def _():
                scat_wait((c - 1) % nb)

            @pl.when(c + 2 < nch)
            def _():
                stage_wait(c + 2, (b + 2) % nb)
                gath(c + 2, (b + 2) % nb)

        def loop3(i, _):
            c0 = i * nb
            for u in range(nb):
                @pl.when(c0 + u < nch)
                def _():
                    chunk(c0 + u, u)

            return 0

        lax.fori_loop(0, (nch + nb - 1) // nb, loop3, 0)
        scat_wait((nch - 1) % nb)
        plsc.subcore_barrier()
        for j in range(jmax):
            c = sid + j * ns

            @pl.when(c < nzch)
            def _():
                pltpu.sync_copy(acc_sh.at[pl.ds(c * zr, zr)],
                                out_acc.at[cid, pl.ds(c * zr, zr)])

    f = pl.kernel(
        body,
        out_type=jax.ShapeDtypeStruct((nc, n, w), jnp.float32),
        mesh=mesh,
        scratch_types=[
            pltpu.VMEM_SHARED((n, w), jnp.float32),
            pltpu.VMEM((3, k), jnp.int32),
            pltpu.VMEM((3, k), jnp.int32),
            pltpu.VMEM((3, k), jnp.int32),
            pltpu.VMEM((3, k, w), jnp.float32),
            pltpu.SemaphoreType.DMA,
            pltpu.SemaphoreType.DMA,
            pltpu.SemaphoreType.DMA,
        ],
    )
    return f(g, src, dst)


def _sc_degree(dst, n):
    e = dst.shape[0]
    w = 128
    mesh = plsc.VectorSubcoreMesh(core_axis_name="c", subcore_axis_name="s")
    nc, ns = mesh.num_cores, mesh.num_subcores
    nw = nc * ns
    epw = e // nw
    zr = k = 80
    nch = epw // k
    nzch = n // zr
    jmax = (nzch + ns - 1) // ns
    wl = w // 16
    assert e % nw == 0 and epw % k == 0 and n % zr == 0

    def body(dst_hbm, out_deg, deg_sh, didx, onesb, semi):
        cid = lax.axis_index("c")
        sid = lax.axis_index("s")
        wid = cid * ns + sid
        base = wid * epw
        zv = jnp.zeros((16,), jnp.float32)
        ov = jnp.ones((16,), jnp.float32)

        def zfill(i, _):
            onesb[i // wl, pl.ds((i % wl) * 16, 16)] = zv
            return 0

        lax.fori_loop(0, k * wl, zfill, 0)
        for j in range(jmax):
            c = sid + j * ns

            @pl.when(c < nzch)
            def _():
                pltpu.sync_copy(onesb, deg_sh.at[pl.ds(c * zr, zr)])

        def ofill(i, _):
            onesb[i // wl, pl.ds((i % wl) * 16, 16)] = ov
            return 0

        lax.fori_loop(0, k * wl, ofill, 0)
        plsc.subcore_barrier()

        def stage(c, b):
            pltpu.async_copy(dst_hbm.at[pl.ds(base + c * k, k)], didx.at[b], semi)

        def stage_wait(c, b):
            pltpu.make_async_copy(
                dst_hbm.at[pl.ds(base + c * k, k)], didx.at[b], semi).wait()

        stage(0, 0)

        def chunk(c, b):
            stage_wait(c, b)

            @pl.when(c + 1 < nch)
            def _():
                stage(c + 1, 1 - b)

            pltpu.sync_copy(onesb, deg_sh.at[didx.at[b]], add=True)

        def loop2(i, _):
            c0 = i * 2

            @pl.when(c0 < nch)
            def _():
                chunk(c0, 0)

            @pl.when(c0 + 1 < nch)
            def _():
                chunk(c0 + 1, 1)

            return 0

        lax.fori_loop(0, (nch + 1) // 2, loop2, 0)
        plsc.subcore_barrier()
        for j in range(jmax):
            c = sid + j * ns

            @pl.when(c < nzch)
            def _():
                pltpu.sync_copy(deg_sh.at[pl.ds(c * zr, zr)],
                                out_deg.at[cid, pl.ds(c * zr, zr)])

    f = pl.kernel(
        body,
        out_type=jax.ShapeDtypeStruct((nc, n, w), jnp.float32),
        mesh=mesh,
        scratch_types=[
            pltpu.VMEM_SHARED((n, w), jnp.float32),
            pltpu.VMEM((2, k), jnp.int32),
            pltpu.VMEM((k, w), jnp.float32),
            pltpu.SemaphoreType.DMA,
        ],
    )
    return f(dst)


def _mm_dual_body(x_ref, wa_ref, wb_ref, a_ref, b_ref):
    x = x_ref[...]
    a_ref[...] = jnp.dot(x, wa_ref[...], preferred_element_type=jnp.float32)
    b_ref[...] = jnp.dot(x, wb_ref[...], preferred_element_type=jnp.float32)


def _mm_dual(x, wa, wb):
    n, d = x.shape
    ha, hb = wa.shape[1], wb.shape[1]
    return pl.pallas_call(
        _mm_dual_body,
        grid=(n // BLK,),
        in_specs=[
            pl.BlockSpec((BLK, d), lambda i: (i, 0)),
            pl.BlockSpec((d, ha), lambda i: (0, 0)),
            pl.BlockSpec((d, hb), lambda i: (0, 0)),
        ],
        out_specs=[
            pl.BlockSpec((BLK, ha), lambda i: (i, 0)),
            pl.BlockSpec((BLK, hb), lambda i: (i, 0)),
        ],
        out_shape=[
            jax.ShapeDtypeStruct((n, ha), jnp.float32),
            jax.ShapeDtypeStruct((n, hb), jnp.float32),
        ],
    )(x, wa, wb)


def _invdeg_body(degp_ref, out_ref):
    d = degp_ref[0] + degp_ref[1]
    e0 = (lax.broadcasted_iota(jnp.int32, (d.shape[1], 1), 0) == 0)
    deg = jnp.dot(d, e0.astype(jnp.float32),
                  preferred_element_type=jnp.float32)
    out_ref[...] = 1.0 / jnp.maximum(deg, 1.0)


def _invdeg(degp):
    nc, n, w = degp.shape
    return pl.pallas_call(
        _invdeg_body,
        grid=(n // BLK,),
        in_specs=[pl.BlockSpec((nc, BLK, w), lambda i: (0, i, 0))],
        out_specs=pl.BlockSpec((BLK, 1), lambda i: (i, 0)),
        out_shape=jax.ShapeDtypeStruct((n, 1), jnp.float32),
    )(degp)


def _mid_body(self0_ref, s0p_ref, invd_ref, b0_ref, h1_ref):
    s0 = s0p_ref[0] + s0p_ref[1]
    h1_ref[...] = jnp.maximum(
        self0_ref[...] + s0 * invd_ref[...] + b0_ref[...], 0.0)


def _mid(self0, s0p, invd, b0):
    n, h = self0.shape
    nc = s0p.shape[0]
    return pl.pallas_call(
        _mid_body,
        grid=(n // BLK,),
        in_specs=[
            pl.BlockSpec((BLK, h), lambda i: (i, 0)),
            pl.BlockSpec((nc, BLK, h), lambda i: (0, i, 0)),
            pl.BlockSpec((BLK, 1), lambda i: (i, 0)),
            pl.BlockSpec((1, h), lambda i: (0, 0)),
        ],
        out_specs=pl.BlockSpec((BLK, h), lambda i: (i, 0)),
        out_shape=jax.ShapeDtypeStruct((n, h), jnp.float32),
    )(self0, s0p, invd, b0)


def _final_body(h1_ref, s1p_ref, invd_ref, b1_ref, w1s_ref, w1n_ref, out_ref):
    hn = (s1p_ref[0] + s1p_ref[1]) * invd_ref[...]
    out_ref[...] = (
        jnp.dot(h1_ref[...], w1s_ref[...], preferred_element_type=jnp.float32)
        + jnp.dot(hn, w1n_ref[...], preferred_element_type=jnp.float32)
        + b1_ref[...])


def _final(h1, s1p, invd, b1, w1s, w1n):
    n, h = h1.shape
    nc = s1p.shape[0]
    c = w1s.shape[1]
    return pl.pallas_call(
        _final_body,
        grid=(n // BLK,),
        in_specs=[
            pl.BlockSpec((BLK, h), lambda i: (i, 0)),
            pl.BlockSpec((nc, BLK, h), lambda i: (0, i, 0)),
            pl.BlockSpec((BLK, 1), lambda i: (i, 0)),
            pl.BlockSpec((1, c), lambda i: (0, 0)),
            pl.BlockSpec((h, c), lambda i: (0, 0)),
            pl.BlockSpec((h, c), lambda i: (0, 0)),
        ],
        out_specs=pl.BlockSpec((BLK, c), lambda i: (i, 0)),
        out_shape=jax.ShapeDtypeStruct((n, c), jnp.float32),
    )(h1, s1p, invd, b1, w1s, w1n)


def kernel(inputs, graph, W0_self, W0_neigh, b0, W1_self, W1_neigh, b1):
    n = inputs.shape[0]
    gi = graph.astype(jnp.int32)
    src, dst = gi[0], gi[1]
    degp = _sc_degree(dst, n)
    invd = _invdeg(degp)
    self0, g0 = _mm_dual(inputs, W0_self, W0_neigh)
    s0p = _sc_edge_scatter(g0, src, dst)
    h1 = _mid(self0, s0p, invd, b0.reshape(1, -1))
    s1p = _sc_edge_scatter(h1, src, dst)
    return _final(h1, s1p, invd, b1.reshape(1, -1), W1_self, W1_neigh)
